# Initial kernel scaffold; baseline (speedup 1.0000x reference)
import jax
import jax.numpy as jnp
from jax.experimental import pallas as pl


def kernel(x, node_type, edge_index, batch, st_table, nt_table, W1, b1, gamma, beta, W2, b2, pool_w, W3, b3):
    raise NotImplementedError("write your pallas kernel here")


# speedup vs baseline: 3.8170x; 3.8170x over previous
import functools

import jax
import jax.numpy as jnp
from jax import lax
from jax.experimental import pallas as pl
from jax.experimental.pallas import tpu as pltpu
from jax.experimental.pallas import tpu_sc as plsc

N = 10000
L = 16
E = 320000
B = 200
NPG = 50
D = 128
H = 128
K = 25
BN_EPS = 1e-5

NC = 2
NS = 16
NW = NC * NS


_NODES_PER_CHUNK = 8
_NCHUNK = N // _NODES_PER_CHUNK


def _emb_body(x_hbm, nt_hbm, st_hbm, ntt_hbm, emb_out, relu_out,
              xidx, ntidx, strows, ntrows, emb_v, relu_v, sem1, sem2):
  wid = lax.axis_index("s") * NC + lax.axis_index("c")
  my_n = (_NCHUNK - 1 - wid) // NW + 1

  def body(i, carry):
    c = wid + i * NW
    pltpu.sync_copy(x_hbm.at[pl.ds(pl.multiple_of(c * 128, 128), 128)], xidx)
    pltpu.sync_copy(nt_hbm.at[pl.ds(pl.multiple_of(c * 8, 8), 8)], ntidx)
    cp1 = pltpu.async_copy(st_hbm.at[xidx], strows, sem1)
    cp2 = pltpu.async_copy(ntt_hbm.at[ntidx], ntrows, sem2)
    cp1.wait()
    cp2.wait()
    for j in range(_NODES_PER_CHUNK):
      ids = xidx[pl.ds(j * 16, 16)]
      cnt = plsc.all_reduce_population_count(ids != 0)
      rec = 1.0 / jnp.maximum(cnt, 1).astype(jnp.float32)
      for k in range(D // 16):
        acc = strows[j * 16, pl.ds(k * 16, 16)]
        for l in range(1, L):
          acc = acc + strows[j * 16 + l, pl.ds(k * 16, 16)]
        val = acc * rec + ntrows[j, pl.ds(k * 16, 16)]
        emb_v[j, pl.ds(k * 16, 16)] = val
        relu_v[j, pl.ds(k * 16, 16)] = jnp.maximum(val, 0.0)
    pltpu.sync_copy(emb_v, emb_out.at[pl.ds(pl.multiple_of(c * 8, 8), 8)])
    pltpu.sync_copy(relu_v, relu_out.at[pl.ds(pl.multiple_of(c * 8, 8), 8)])
    return carry

  lax.fori_loop(0, my_n, body, 0)


def _sc_embed(x_flat, node_type, st_table, nt_table):
  f = pl.kernel(
      _emb_body,
      out_type=(jax.ShapeDtypeStruct((N, D), jnp.float32),
                jax.ShapeDtypeStruct((N, D), jnp.float32)),
      mesh=plsc.VectorSubcoreMesh(core_axis_name="c", subcore_axis_name="s"),
      compiler_params=pltpu.CompilerParams(needs_layout_passes=False),
      scratch_types=[
          pltpu.VMEM((128,), jnp.int32),
          pltpu.VMEM((_NODES_PER_CHUNK,), jnp.int32),
          pltpu.VMEM((128, D), jnp.float32),
          pltpu.VMEM((_NODES_PER_CHUNK, D), jnp.float32),
          pltpu.VMEM((_NODES_PER_CHUNK, D), jnp.float32),
          pltpu.VMEM((_NODES_PER_CHUNK, D), jnp.float32),
          pltpu.SemaphoreType.DMA,
          pltpu.SemaphoreType.DMA,
      ],
  )
  return f(x_flat, node_type, st_table, nt_table)


_ECH = 80
_EPW = E // NW
_ZROWS = 200
_NZCH = N // _ZROWS


def _edge_body(src_hbm, dst_hbm, remb_hbm, agg_out,
               sidx, didx, rows, zbuf, agg_sh, sem):
  cid = lax.axis_index("c")
  sid = lax.axis_index("s")
  wid = sid * NC + cid
  my_z = (_NZCH - 1 - sid) // NS + 1

  zeros16 = jnp.zeros((16,), jnp.float32)

  def zb(r, carry):
    for k in range(D // 16):
      zbuf[r, pl.ds(k * 16, 16)] = zeros16
    return carry

  lax.fori_loop(0, _ZROWS, zb, 0)

  def zcopy(i, carry):
    off = pl.multiple_of((sid + i * NS) * _ZROWS, 8)
    pltpu.sync_copy(zbuf, agg_sh.at[pl.ds(off, _ZROWS)])
    return carry

  lax.fori_loop(0, my_z, zcopy, 0)
  plsc.subcore_barrier()

  def eb(i, carry):
    b = pl.multiple_of(wid * _EPW + i * _ECH, 8)
    pltpu.sync_copy(src_hbm.at[pl.ds(b, _ECH)], sidx)
    pltpu.sync_copy(dst_hbm.at[pl.ds(b, _ECH)], didx)
    pltpu.async_copy(remb_hbm.at[sidx], rows, sem).wait()
    pltpu.sync_copy(rows, agg_sh.at[didx], add=True)
    return carry

  lax.fori_loop(0, _EPW // _ECH, eb, 0)
  plsc.subcore_barrier()

  def wcopy(i, carry):
    off = pl.multiple_of((sid + i * NS) * _ZROWS, 8)
    pltpu.sync_copy(agg_sh.at[pl.ds(off, _ZROWS)],
                    agg_out.at[pl.ds(cid * N + off, _ZROWS)])
    return carry

  lax.fori_loop(0, my_z, wcopy, 0)


def _sc_edges(src, dst, relu_emb):
  f = pl.kernel(
      _edge_body,
      out_type=jax.ShapeDtypeStruct((NC * N, D), jnp.float32),
      mesh=plsc.VectorSubcoreMesh(core_axis_name="c", subcore_axis_name="s"),
      compiler_params=pltpu.CompilerParams(needs_layout_passes=False),
      scratch_types=[
          pltpu.VMEM((_ECH,), jnp.int32),
          pltpu.VMEM((_ECH,), jnp.int32),
          pltpu.VMEM((_ECH, D), jnp.float32),
          pltpu.VMEM((_ZROWS, D), jnp.float32),
          pltpu.VMEM_SHARED((N, D), jnp.float32),
          pltpu.SemaphoreType.DMA,
      ],
  )
  return f(src, dst, relu_emb)


def _tc_body(emb_ref, agg_ref, w1_ref, b1_ref, g_ref, be_ref, w2_ref, b2_ref,
             wc_ref, w3_ref, b3_ref, out_ref):
  h = emb_ref[...] + agg_ref[0:N, :] + agg_ref[N:2 * N, :]
  h1 = jnp.dot(h, w1_ref[...], preferred_element_type=jnp.float32) + b1_ref[...]
  mu = jnp.sum(h1, axis=0, keepdims=True) * (1.0 / N)
  dlt = h1 - mu
  var = jnp.sum(dlt * dlt, axis=0, keepdims=True) * (1.0 / N)
  h1n = dlt * lax.rsqrt(var + BN_EPS) * g_ref[...] + be_ref[...]
  h1n = jnp.maximum(h1n, 0.0)
  xh = jnp.dot(h1n, w2_ref[...], preferred_element_type=jnp.float32) + b2_ref[...]

  wc = wc_ref[...]
  inv_norm = lax.rsqrt(jnp.sum(wc * wc))
  s_flat = jnp.dot(xh, wc, preferred_element_type=jnp.float32) * inv_norm

  s2 = jnp.concatenate([s_flat[n * B:(n + 1) * B, :] for n in range(NPG)],
                       axis=1)
  lane = lax.broadcasted_iota(jnp.int32, (B, NPG), 1)
  rank = jnp.zeros((B, NPG), jnp.int32)
  for m in range(NPG):
    cm = s2[:, m:m + 1]
    rank = rank + jnp.where(cm > s2, 1, 0) \
                + jnp.where((cm == s2) & (lane > m), 1, 0)
  sel = rank < K
  t2 = jnp.tanh(s2)

  neg = jnp.float32(-3.0e38)
  gmax = jnp.full((B, H), neg, jnp.float32)
  gsum = jnp.zeros((B, H), jnp.float32)
  for n in range(NPG):
    xn = xh[n * B:(n + 1) * B, :]
    xpn = xn * t2[:, n:n + 1]
    mn = sel[:, n:n + 1]
    gmax = jnp.maximum(gmax, jnp.where(mn, xpn, neg))
    gsum = gsum + jnp.where(mn, xpn, 0.0)
  stmt = jnp.concatenate([gmax, gsum * (1.0 / K)], axis=1)
  out_ref[...] = jnp.dot(stmt, w3_ref[...],
                         preferred_element_type=jnp.float32) + b3_ref[...]


def _tc_dense(emb, aggp, W1, b1, gamma, beta, W2, b2, pool_w, W3, b3):
  return pl.pallas_call(
      _tc_body,
      out_shape=jax.ShapeDtypeStruct((B, H), jnp.float32),
  )(emb, aggp, W1, b1.reshape(1, -1), gamma.reshape(1, -1),
    beta.reshape(1, -1), W2, b2.reshape(1, -1), pool_w.reshape(D, 1),
    W3, b3.reshape(1, -1))


def kernel(x, node_type, edge_index, batch, st_table, nt_table, W1, b1,
           gamma, beta, W2, b2, pool_w, W3, b3):
  del batch
  r = jnp.arange(N, dtype=jnp.int32)
  inv = (r % B) * NPG + r // B
  x_flat = x[inv].astype(jnp.int32).reshape(N * L)
  nt_p = node_type[inv].astype(jnp.int32)
  src = edge_index[0].astype(jnp.int32)
  dst = edge_index[1].astype(jnp.int32)
  src_p = (src % NPG) * B + src // NPG
  dst_p = (dst % NPG) * B + dst // NPG

  emb, relu_emb = _sc_embed(x_flat, nt_p, st_table, nt_table)
  aggp = _sc_edges(src_p, dst_p, relu_emb)
  return _tc_dense(emb, aggp, W1, b1, gamma, beta, W2, b2, pool_w, W3, b3)


# speedup vs baseline: 4.8094x; 1.2600x over previous
import jax
import jax.numpy as jnp
from jax import lax
from jax.experimental import pallas as pl
from jax.experimental.pallas import tpu as pltpu
from jax.experimental.pallas import tpu_sc as plsc

N = 10000
L = 16
E = 320000
B = 200
NPG = 50
D = 128
H = 128
K = 25
BN_EPS = 1e-5

NC = 2
NS = 16
NW = NC * NS


_CPN = 8
_CHW = 40
_NPAD = NW * _CHW * _CPN


def _emb_body(x_hbm, nt_hbm, st_hbm, ntt_hbm, emb_out, relu_out,
              xidx_all, ntidx_all, strows2, ntrows2, embw, reluw,
              semg0, semg1, semn0, semn1, semw0, semw1):
  wid = lax.axis_index("s") * NC + lax.axis_index("c")
  semg = (semg0, semg1)
  semn = (semn0, semn1)
  semw = (semw0, semw1)

  pltpu.sync_copy(x_hbm.at[pl.ds(pl.multiple_of(wid * _CHW, 8), _CHW)],
                  xidx_all)
  pltpu.sync_copy(nt_hbm.at[pl.ds(pl.multiple_of(wid * _CHW, 8), _CHW)],
                  ntidx_all)

  def out_row(i):
    return pl.ds(pl.multiple_of(wid * (_CHW * _CPN) + i * _CPN, 8), _CPN)

  def issue(i, s):
    pltpu.async_copy(st_hbm.at[xidx_all.at[i]], strows2.at[s], semg[s])
    pltpu.async_copy(ntt_hbm.at[ntidx_all.at[i]], ntrows2.at[s], semn[s])

  def wait_g(i, s):
    pltpu.make_async_copy(st_hbm.at[xidx_all.at[i]], strows2.at[s],
                          semg[s]).wait()
    pltpu.make_async_copy(ntt_hbm.at[ntidx_all.at[i]], ntrows2.at[s],
                          semn[s]).wait()

  def drain_w(i_old, s):
    pltpu.make_async_copy(embw.at[s], emb_out.at[out_row(i_old)],
                          semw[s]).wait()
    pltpu.make_async_copy(reluw.at[s], relu_out.at[out_row(i_old)],
                          semw[s]).wait()

  def compute(i, s):
    for j in range(_CPN):
      ids = xidx_all[i, pl.ds(j * 16, 16)]
      cnt = plsc.all_reduce_population_count(ids != 0)
      rec = 1.0 / jnp.maximum(cnt, 1).astype(jnp.float32)
      for k in range(D // 16):
        acc = strows2[s, j * 16, pl.ds(k * 16, 16)]
        for l in range(1, L):
          acc = acc + strows2[s, j * 16 + l, pl.ds(k * 16, 16)]
        val = acc * rec + ntrows2[s, j, pl.ds(k * 16, 16)]
        embw[s, j, pl.ds(k * 16, 16)] = val
        reluw[s, j, pl.ds(k * 16, 16)] = jnp.maximum(val, 0.0)
    pltpu.async_copy(embw.at[s], emb_out.at[out_row(i)], semw[s])
    pltpu.async_copy(reluw.at[s], relu_out.at[out_row(i)], semw[s])

  issue(0, 0)

  def body(j, carry):
    i0 = 2 * j
    i1 = 2 * j + 1
    issue(i1, 1)
    wait_g(i0, 0)

    @pl.when(j > 0)
    def _():
      drain_w(i0 - 2, 0)

    compute(i0, 0)

    @pl.when(j < _CHW // 2 - 1)
    def _():
      issue(i0 + 2, 0)

    @pl.when(j > 0)
    def _():
      drain_w(i1 - 2, 1)

    wait_g(i1, 1)
    compute(i1, 1)
    return carry

  lax.fori_loop(0, _CHW // 2, body, 0)
  drain_w(_CHW - 2, 0)
  drain_w(_CHW - 1, 1)


def _sc_embed(x3, nt3, st_table, nt_table):
  f = pl.kernel(
      _emb_body,
      out_type=(jax.ShapeDtypeStruct((_NPAD, D), jnp.float32),
                jax.ShapeDtypeStruct((_NPAD, D), jnp.float32)),
      mesh=plsc.VectorSubcoreMesh(core_axis_name="c", subcore_axis_name="s"),
      compiler_params=pltpu.CompilerParams(needs_layout_passes=False),
      scratch_types=[
          pltpu.VMEM((_CHW, _CPN * L), jnp.int32),
          pltpu.VMEM((_CHW, _CPN), jnp.int32),
          pltpu.VMEM((2, _CPN * L, D), jnp.float32),
          pltpu.VMEM((2, _CPN, D), jnp.float32),
          pltpu.VMEM((2, _CPN, D), jnp.float32),
          pltpu.VMEM((2, _CPN, D), jnp.float32),
          pltpu.SemaphoreType.DMA,
          pltpu.SemaphoreType.DMA,
          pltpu.SemaphoreType.DMA,
          pltpu.SemaphoreType.DMA,
          pltpu.SemaphoreType.DMA,
          pltpu.SemaphoreType.DMA,
      ],
  )
  return f(x3, nt3, st_table, nt_table)


_ECH = 125
_ECHN = 80
_ZROWS = 200
_NZCH = N // _ZROWS


def _edge_body(src_hbm, dst_hbm, remb_hbm, agg_out,
               sidx_all, didx_all, rows2, agg_sh, semg0, semg1):
  cid = lax.axis_index("c")
  sid = lax.axis_index("s")
  wid = sid * NC + cid
  my_z = (_NZCH - 1 - sid) // NS + 1
  semg = (semg0, semg1)

  zeros16 = jnp.zeros((16,), jnp.float32)

  def zb(r, carry):
    for k in range(D // 16):
      rows2[0, r, pl.ds(k * 16, 16)] = zeros16
    return carry

  lax.fori_loop(0, 128, zb, 0)

  my_zc = (78 - sid) // NS + 1

  def zcopy(i, carry):
    off = pl.multiple_of(jnp.minimum((sid + i * NS) * 128, N - 128), 8)
    pltpu.sync_copy(rows2.at[0], agg_sh.at[pl.ds(off, 128)])
    return carry

  lax.fori_loop(0, my_zc, zcopy, 0)
  plsc.subcore_barrier()

  def start(i, s):
    pltpu.async_copy(remb_hbm.at[sidx_all.at[i]],
                     rows2.at[s, pl.ds(0, _ECH)], semg[s])

  def wait(i, s):
    pltpu.make_async_copy(remb_hbm.at[sidx_all.at[i]],
                          rows2.at[s, pl.ds(0, _ECH)], semg[s]).wait()

  for half in range(2):
    pltpu.sync_copy(src_hbm.at[wid, half], sidx_all)
    pltpu.sync_copy(dst_hbm.at[wid, half], didx_all)
    start(0, 0)

    def eb(j, carry):
      i0 = 2 * j
      i1 = 2 * j + 1
      start(i1, 1)
      wait(i0, 0)
      pltpu.sync_copy(rows2.at[0, pl.ds(0, _ECH)],
                      agg_sh.at[didx_all.at[i0]], add=True)

      @pl.when(j < _ECHN // 4 - 1)
      def _():
        start(i0 + 2, 0)

      wait(i1, 1)
      pltpu.sync_copy(rows2.at[1, pl.ds(0, _ECH)],
                      agg_sh.at[didx_all.at[i1]], add=True)
      return carry

    lax.fori_loop(0, _ECHN // 4, eb, 0)
  plsc.subcore_barrier()

  def wcopy(i, carry):
    off = pl.multiple_of((sid + i * NS) * _ZROWS, 8)
    pltpu.sync_copy(agg_sh.at[pl.ds(off, _ZROWS)],
                    agg_out.at[pl.ds(cid * N + off, _ZROWS)])
    return carry

  lax.fori_loop(0, my_z, wcopy, 0)


def _sc_edges(src3, dst3, relu_emb):
  f = pl.kernel(
      _edge_body,
      out_type=jax.ShapeDtypeStruct((NC * N, D), jnp.float32),
      mesh=plsc.VectorSubcoreMesh(core_axis_name="c", subcore_axis_name="s"),
      compiler_params=pltpu.CompilerParams(needs_layout_passes=False),
      scratch_types=[
          pltpu.VMEM((_ECHN // 2, _ECH), jnp.int32),
          pltpu.VMEM((_ECHN // 2, _ECH), jnp.int32),
          pltpu.VMEM((2, 128, D), jnp.float32),
          pltpu.VMEM_SHARED((N, D), jnp.float32),
          pltpu.SemaphoreType.DMA,
          pltpu.SemaphoreType.DMA,
      ],
  )
  return f(src3, dst3, relu_emb)


def _tc_body(emb_ref, agg_ref, w1_ref, b1_ref, g_ref, be_ref, w2_ref, b2_ref,
             wc_ref, w3_ref, b3_ref, out_ref):
  h = emb_ref[...] + agg_ref[0:N, :] + agg_ref[N:2 * N, :]
  h1 = jnp.dot(h, w1_ref[...], preferred_element_type=jnp.float32) + b1_ref[...]
  mu = jnp.sum(h1, axis=0, keepdims=True) * (1.0 / N)
  dlt = h1 - mu
  var = jnp.sum(dlt * dlt, axis=0, keepdims=True) * (1.0 / N)
  h1n = dlt * lax.rsqrt(var + BN_EPS) * g_ref[...] + be_ref[...]
  h1n = jnp.maximum(h1n, 0.0)
  xh = jnp.dot(h1n, w2_ref[...], preferred_element_type=jnp.float32) + b2_ref[...]

  wc = wc_ref[...]
  inv_norm = lax.rsqrt(jnp.sum(wc * wc))
  s_flat = jnp.dot(xh, wc, preferred_element_type=jnp.float32) * inv_norm

  s2 = jnp.concatenate([s_flat[n * B:(n + 1) * B, :] for n in range(NPG)],
                       axis=1)
  lane = lax.broadcasted_iota(jnp.int32, (B, NPG), 1)
  rank = jnp.zeros((B, NPG), jnp.int32)
  for m in range(NPG):
    cm = s2[:, m:m + 1]
    rank = rank + jnp.where(cm > s2, 1, 0) \
                + jnp.where((cm == s2) & (lane > m), 1, 0)
  sel = rank < K
  t2 = jnp.tanh(s2)

  neg = jnp.float32(-3.0e38)
  gmax = jnp.full((B, H), neg, jnp.float32)
  gsum = jnp.zeros((B, H), jnp.float32)
  for n in range(NPG):
    xn = xh[n * B:(n + 1) * B, :]
    xpn = xn * t2[:, n:n + 1]
    mn = sel[:, n:n + 1]
    gmax = jnp.maximum(gmax, jnp.where(mn, xpn, neg))
    gsum = gsum + jnp.where(mn, xpn, 0.0)
  stmt = jnp.concatenate([gmax, gsum * (1.0 / K)], axis=1)
  out_ref[...] = jnp.dot(stmt, w3_ref[...],
                         preferred_element_type=jnp.float32) + b3_ref[...]


def _tc_dense(emb, aggp, W1, b1, gamma, beta, W2, b2, pool_w, W3, b3):
  return pl.pallas_call(
      _tc_body,
      out_shape=jax.ShapeDtypeStruct((B, H), jnp.float32),
  )(emb, aggp, W1, b1.reshape(1, -1), gamma.reshape(1, -1),
    beta.reshape(1, -1), W2, b2.reshape(1, -1), pool_w.reshape(D, 1),
    W3, b3.reshape(1, -1))


def kernel(x, node_type, edge_index, batch, st_table, nt_table, W1, b1,
           gamma, beta, W2, b2, pool_w, W3, b3):
  del batch
  r = jnp.arange(N, dtype=jnp.int32)
  inv = (r % B) * NPG + r // B
  x_flat = x[inv].astype(jnp.int32).reshape(N * L)
  x3 = jnp.concatenate(
      [x_flat, jnp.zeros((_NPAD * L - N * L,), jnp.int32)]).reshape(
          NW * _CHW, _CPN * L)
  nt_flat = node_type[inv].astype(jnp.int32)
  nt3 = jnp.concatenate(
      [nt_flat, jnp.zeros((_NPAD - N,), jnp.int32)]).reshape(
          NW * _CHW, _CPN)
  src = edge_index[0].astype(jnp.int32)
  dst = edge_index[1].astype(jnp.int32)
  src3 = ((src % NPG) * B + src // NPG).reshape(NW, 2, _ECHN // 2, _ECH)
  dst3 = ((dst % NPG) * B + dst // NPG).reshape(NW, 2, _ECHN // 2, _ECH)

  emb, relu_emb = _sc_embed(x3, nt3, st_table, nt_table)
  aggp = _sc_edges(src3, dst3, relu_emb)
  return _tc_dense(emb[:N], aggp, W1, b1, gamma, beta, W2, b2, pool_w, W3, b3)


# speedup vs baseline: 8.5056x; 1.7685x over previous
import jax
import jax.numpy as jnp
from jax import lax
from jax.experimental import pallas as pl
from jax.experimental.pallas import tpu as pltpu
from jax.experimental.pallas import tpu_sc as plsc

N = 10000
L = 16
E = 320000
B = 200
NPG = 50
D = 128
H = 128
K = 25
BN_EPS = 1e-5

NC = 2
NS = 16
NW = NC * NS


_CPN = 8
_CHW = 40
_NPAD = NW * _CHW * _CPN


def _emb_body(x_hbm, nt_hbm, perm_hbm, st_hbm, ntt_hbm, emb_out, relu_out,
              xidx_all, ntidx_all, pidx_all, strows2, ntrows2, embw, reluw,
              semg0, semg1, semn0, semn1, semw0, semw1):
  wid = lax.axis_index("s") * NC + lax.axis_index("c")
  semg = (semg0, semg1)
  semn = (semn0, semn1)
  semw = (semw0, semw1)

  pltpu.sync_copy(x_hbm.at[pl.ds(pl.multiple_of(wid * _CHW, 8), _CHW)],
                  xidx_all)
  pltpu.sync_copy(nt_hbm.at[pl.ds(pl.multiple_of(wid * _CHW, 8), _CHW)],
                  ntidx_all)
  pltpu.sync_copy(perm_hbm.at[pl.ds(pl.multiple_of(wid * _CHW, 8), _CHW)],
                  pidx_all)

  def issue(i, s):
    pltpu.async_copy(st_hbm.at[xidx_all.at[i]], strows2.at[s], semg[s])
    pltpu.async_copy(ntt_hbm.at[ntidx_all.at[i]], ntrows2.at[s], semn[s])

  def wait_g(i, s):
    pltpu.make_async_copy(st_hbm.at[xidx_all.at[i]], strows2.at[s],
                          semg[s]).wait()
    pltpu.make_async_copy(ntt_hbm.at[ntidx_all.at[i]], ntrows2.at[s],
                          semn[s]).wait()

  def drain_w(i_old, s):
    pltpu.make_async_copy(embw.at[s], emb_out.at[pidx_all.at[i_old]],
                          semw[s]).wait()
    pltpu.make_async_copy(reluw.at[s], relu_out.at[pidx_all.at[i_old]],
                          semw[s]).wait()

  def compute(i, s):
    for j in range(_CPN):
      ids = xidx_all[i, pl.ds(j * 16, 16)]
      cnt = plsc.all_reduce_population_count(ids != 0)
      rec = 1.0 / jnp.maximum(cnt, 1).astype(jnp.float32)
      for k in range(D // 16):
        acc = strows2[s, j * 16, pl.ds(k * 16, 16)]
        for l in range(1, L):
          acc = acc + strows2[s, j * 16 + l, pl.ds(k * 16, 16)]
        val = acc * rec + ntrows2[s, j, pl.ds(k * 16, 16)]
        embw[s, j, pl.ds(k * 16, 16)] = val
        reluw[s, j, pl.ds(k * 16, 16)] = jnp.maximum(val, 0.0)
    pltpu.async_copy(embw.at[s], emb_out.at[pidx_all.at[i]], semw[s])
    pltpu.async_copy(reluw.at[s], relu_out.at[pidx_all.at[i]], semw[s])

  issue(0, 0)

  def body(j, carry):
    i0 = 2 * j
    i1 = 2 * j + 1
    issue(i1, 1)
    wait_g(i0, 0)

    @pl.when(j > 0)
    def _():
      drain_w(i0 - 2, 0)

    compute(i0, 0)

    @pl.when(j < _CHW // 2 - 1)
    def _():
      issue(i0 + 2, 0)

    @pl.when(j > 0)
    def _():
      drain_w(i1 - 2, 1)

    wait_g(i1, 1)
    compute(i1, 1)
    return carry

  lax.fori_loop(0, _CHW // 2, body, 0)
  drain_w(_CHW - 2, 0)
  drain_w(_CHW - 1, 1)


def _sc_embed(x3, nt3, perm3, st_table, nt_table):
  f = pl.kernel(
      _emb_body,
      out_type=(jax.ShapeDtypeStruct((_NPAD, D), jnp.float32),
                jax.ShapeDtypeStruct((_NPAD, D), jnp.float32)),
      mesh=plsc.VectorSubcoreMesh(core_axis_name="c", subcore_axis_name="s"),
      compiler_params=pltpu.CompilerParams(needs_layout_passes=False),
      scratch_types=[
          pltpu.VMEM((_CHW, _CPN * L), jnp.int32),
          pltpu.VMEM((_CHW, _CPN), jnp.int32),
          pltpu.VMEM((_CHW, _CPN), jnp.int32),
          pltpu.VMEM((2, _CPN * L, D), jnp.float32),
          pltpu.VMEM((2, _CPN, D), jnp.float32),
          pltpu.VMEM((2, _CPN, D), jnp.float32),
          pltpu.VMEM((2, _CPN, D), jnp.float32),
          pltpu.SemaphoreType.DMA,
          pltpu.SemaphoreType.DMA,
          pltpu.SemaphoreType.DMA,
          pltpu.SemaphoreType.DMA,
          pltpu.SemaphoreType.DMA,
          pltpu.SemaphoreType.DMA,
      ],
  )
  return f(x3, nt3, perm3, st_table, nt_table)


_ECH = 125
_ECHN = 80
_ZROWS = 200
_NZCH = N // _ZROWS


def _edge_body(src_hbm, dst_hbm, remb_hbm, agg_out,
               sidx_all, didx_all, rows2, agg_sh, semg0, semg1):
  cid = lax.axis_index("c")
  sid = lax.axis_index("s")
  wid = sid * NC + cid
  my_z = (_NZCH - 1 - sid) // NS + 1
  semg = (semg0, semg1)

  zeros16 = jnp.zeros((16,), jnp.float32)

  def zb(r, carry):
    for k in range(D // 16):
      rows2[0, r, pl.ds(k * 16, 16)] = zeros16
    return carry

  lax.fori_loop(0, 128, zb, 0)

  my_zc = (78 - sid) // NS + 1

  def zcopy(i, carry):
    off = pl.multiple_of(jnp.minimum((sid + i * NS) * 128, N - 128), 8)
    pltpu.sync_copy(rows2.at[0], agg_sh.at[pl.ds(off, 128)])
    return carry

  lax.fori_loop(0, my_zc, zcopy, 0)
  plsc.subcore_barrier()

  def start(i, s):
    pltpu.async_copy(remb_hbm.at[sidx_all.at[i]],
                     rows2.at[s, pl.ds(0, _ECH)], semg[s])

  def wait(i, s):
    pltpu.make_async_copy(remb_hbm.at[sidx_all.at[i]],
                          rows2.at[s, pl.ds(0, _ECH)], semg[s]).wait()

  for half in range(2):
    pltpu.sync_copy(src_hbm.at[wid, half], sidx_all)
    pltpu.sync_copy(dst_hbm.at[wid, half], didx_all)
    start(0, 0)

    def eb(j, carry):
      i0 = 2 * j
      i1 = 2 * j + 1
      start(i1, 1)
      wait(i0, 0)
      pltpu.sync_copy(rows2.at[0, pl.ds(0, _ECH)],
                      agg_sh.at[didx_all.at[i0]], add=True)

      @pl.when(j < _ECHN // 4 - 1)
      def _():
        start(i0 + 2, 0)

      wait(i1, 1)
      pltpu.sync_copy(rows2.at[1, pl.ds(0, _ECH)],
                      agg_sh.at[didx_all.at[i1]], add=True)
      return carry

    lax.fori_loop(0, _ECHN // 4, eb, 0)
  plsc.subcore_barrier()

  def wcopy(i, carry):
    off = pl.multiple_of((sid + i * NS) * _ZROWS, 8)
    pltpu.sync_copy(agg_sh.at[pl.ds(off, _ZROWS)],
                    agg_out.at[pl.ds(cid * N + off, _ZROWS)])
    return carry

  lax.fori_loop(0, my_z, wcopy, 0)


def _sc_edges(src3, dst3, relu_emb):
  f = pl.kernel(
      _edge_body,
      out_type=jax.ShapeDtypeStruct((NC * N, D), jnp.float32),
      mesh=plsc.VectorSubcoreMesh(core_axis_name="c", subcore_axis_name="s"),
      compiler_params=pltpu.CompilerParams(needs_layout_passes=False),
      scratch_types=[
          pltpu.VMEM((_ECHN // 2, _ECH), jnp.int32),
          pltpu.VMEM((_ECHN // 2, _ECH), jnp.int32),
          pltpu.VMEM((2, 128, D), jnp.float32),
          pltpu.VMEM_SHARED((N, D), jnp.float32),
          pltpu.SemaphoreType.DMA,
          pltpu.SemaphoreType.DMA,
      ],
  )
  return f(src3, dst3, relu_emb)


def _tc_body(emb_ref, agg_ref, w1_ref, b1_ref, g_ref, be_ref, w2_ref, b2_ref,
             wc_ref, w3_ref, b3_ref, out_ref):
  h = emb_ref[0:N, :] + agg_ref[0:N, :] + agg_ref[N:2 * N, :]
  h1 = jnp.dot(h, w1_ref[...], preferred_element_type=jnp.float32) + b1_ref[...]
  mu = jnp.sum(h1, axis=0, keepdims=True) * (1.0 / N)
  dlt = h1 - mu
  var = jnp.sum(dlt * dlt, axis=0, keepdims=True) * (1.0 / N)
  h1n = dlt * lax.rsqrt(var + BN_EPS) * g_ref[...] + be_ref[...]
  h1n = jnp.maximum(h1n, 0.0)
  xh = jnp.dot(h1n, w2_ref[...], preferred_element_type=jnp.float32) + b2_ref[...]

  wc = wc_ref[...]
  inv_norm = lax.rsqrt(jnp.sum(wc * wc))
  s_flat = jnp.dot(xh, wc, preferred_element_type=jnp.float32) * inv_norm

  s2 = jnp.concatenate([s_flat[n * B:(n + 1) * B, :] for n in range(NPG)],
                       axis=1)
  lane = lax.broadcasted_iota(jnp.int32, (B, NPG), 1)
  rank = jnp.zeros((B, NPG), jnp.int32)
  for m in range(NPG):
    cm = s2[:, m:m + 1]
    rank = rank + jnp.where(cm > s2, 1, 0) \
                + jnp.where((cm == s2) & (lane > m), 1, 0)
  sel = rank < K
  t2 = jnp.tanh(s2)

  neg = jnp.float32(-3.0e38)
  gmax = jnp.full((B, H), neg, jnp.float32)
  gsum = jnp.zeros((B, H), jnp.float32)
  for n in range(NPG):
    xn = xh[n * B:(n + 1) * B, :]
    xpn = xn * t2[:, n:n + 1]
    mn = sel[:, n:n + 1]
    gmax = jnp.maximum(gmax, jnp.where(mn, xpn, neg))
    gsum = gsum + jnp.where(mn, xpn, 0.0)
  stmt = jnp.concatenate([gmax, gsum * (1.0 / K)], axis=1)
  out_ref[...] = jnp.dot(stmt, w3_ref[...],
                         preferred_element_type=jnp.float32) + b3_ref[...]


def _tc_dense(emb, aggp, W1, b1, gamma, beta, W2, b2, pool_w, W3, b3):
  return pl.pallas_call(
      _tc_body,
      out_shape=jax.ShapeDtypeStruct((B, H), jnp.float32),
  )(emb, aggp, W1, b1.reshape(1, -1), gamma.reshape(1, -1),
    beta.reshape(1, -1), W2, b2.reshape(1, -1), pool_w.reshape(D, 1),
    W3, b3.reshape(1, -1))


def kernel(x, node_type, edge_index, batch, st_table, nt_table, W1, b1,
           gamma, beta, W2, b2, pool_w, W3, b3):
  del batch
  x_flat = x.astype(jnp.int32).reshape(N * L)
  pad_ids = (jnp.arange(_NPAD * L - N * L, dtype=jnp.int32) * 997) % 99991
  x3 = jnp.concatenate([x_flat, pad_ids]).reshape(NW * _CHW, _CPN * L)
  nt3 = jnp.concatenate(
      [node_type.astype(jnp.int32), jnp.zeros((_NPAD - N,), jnp.int32)]
  ).reshape(NW * _CHW, _CPN)
  v = jnp.arange(_NPAD, dtype=jnp.int32)
  perm = jnp.where(v < N, (v % NPG) * B + v // NPG, v)
  perm3 = perm.reshape(NW * _CHW, _CPN)
  src = edge_index[0].astype(jnp.int32)
  dst = edge_index[1].astype(jnp.int32)
  src3 = ((src % NPG) * B + src // NPG).reshape(NW, 2, _ECHN // 2, _ECH)
  dst3 = ((dst % NPG) * B + dst // NPG).reshape(NW, 2, _ECHN // 2, _ECH)

  emb, relu_emb = _sc_embed(x3, nt3, perm3, st_table, nt_table)
  aggp = _sc_edges(src3, dst3, relu_emb)
  return _tc_dense(emb, aggp, W1, b1, gamma, beta, W2, b2, pool_w, W3, b3)


# speedup vs baseline: 8.9090x; 1.0474x over previous
import jax
import jax.numpy as jnp
from jax import lax
from jax.experimental import pallas as pl
from jax.experimental.pallas import tpu as pltpu
from jax.experimental.pallas import tpu_sc as plsc

N = 10000
L = 16
E = 320000
B = 200
NPG = 50
D = 128
H = 128
K = 25
BN_EPS = 1e-5

NC = 2
NS = 16
NW = NC * NS


_CPN = 8
_CHW = 40
_NPAD = NW * _CHW * _CPN


def _emb_body(x_hbm, nt_hbm, perm_hbm, st_hbm, ntt_hbm, emb_out, relu_out,
              xidx_all, ntidx_all, pidx_all, strows2, ntt_l, embw, reluw,
              semg0, semg1, semw0, semw1):
  wid = lax.axis_index("s") * NC + lax.axis_index("c")
  semg = (semg0, semg1)
  semw = (semw0, semw1)

  pltpu.sync_copy(x_hbm.at[pl.ds(pl.multiple_of(wid * _CHW, 8), _CHW)],
                  xidx_all)
  pltpu.sync_copy(nt_hbm.at[pl.ds(pl.multiple_of(wid * _CHW, 8), _CHW)],
                  ntidx_all)
  pltpu.sync_copy(perm_hbm.at[pl.ds(pl.multiple_of(wid * _CHW, 8), _CHW)],
                  pidx_all)
  pltpu.sync_copy(ntt_hbm, ntt_l)

  iota16 = lax.broadcasted_iota(jnp.int32, (16,), 0)

  def issue(i, s):
    pltpu.async_copy(st_hbm.at[xidx_all.at[i]], strows2.at[s], semg[s])

  def wait_g(i, s):
    pltpu.make_async_copy(st_hbm.at[xidx_all.at[i]], strows2.at[s],
                          semg[s]).wait()

  def drain_w(i_old, s):
    pltpu.make_async_copy(embw.at[s], emb_out.at[pidx_all.at[i_old]],
                          semw[s]).wait()
    pltpu.make_async_copy(reluw.at[s], relu_out.at[pidx_all.at[i_old]],
                          semw[s]).wait()

  def compute(i, s):
    ntids = ntidx_all[i, pl.ds(0, 16)]
    for j in range(_CPN):
      ids = xidx_all[i, pl.ds(j * 16, 16)]
      cnt = plsc.all_reduce_population_count(ids != 0)
      rec = 1.0 / jnp.maximum(cnt, 1).astype(jnp.float32)
      ntid = ntids.at[jnp.full((16,), j, jnp.int32)].get(
          mode="promise_in_bounds")
      ntbase = ntid * D + iota16
      for k in range(D // 16):
        acc = strows2[s, j * 16, pl.ds(k * 16, 16)]
        for l in range(1, L):
          acc = acc + strows2[s, j * 16 + l, pl.ds(k * 16, 16)]
        val = acc * rec + plsc.load_gather(ntt_l, [ntbase + (k * 16)])
        embw[s, j, pl.ds(k * 16, 16)] = val
        reluw[s, j, pl.ds(k * 16, 16)] = jnp.maximum(val, 0.0)
    pltpu.async_copy(embw.at[s], emb_out.at[pidx_all.at[i]], semw[s])
    pltpu.async_copy(reluw.at[s], relu_out.at[pidx_all.at[i]], semw[s])

  issue(0, 0)

  def body(j, carry):
    i0 = 2 * j
    i1 = 2 * j + 1
    issue(i1, 1)
    wait_g(i0, 0)

    @pl.when(j > 0)
    def _():
      drain_w(i0 - 2, 0)

    compute(i0, 0)

    @pl.when(j < _CHW // 2 - 1)
    def _():
      issue(i0 + 2, 0)

    @pl.when(j > 0)
    def _():
      drain_w(i1 - 2, 1)

    wait_g(i1, 1)
    compute(i1, 1)
    return carry

  lax.fori_loop(0, _CHW // 2, body, 0)
  drain_w(_CHW - 2, 0)
  drain_w(_CHW - 1, 1)


def _sc_embed(x3, nt3, perm3, st_table, nt_table):
  f = pl.kernel(
      _emb_body,
      out_type=(jax.ShapeDtypeStruct((_NPAD, D), jnp.float32),
                jax.ShapeDtypeStruct((_NPAD, D), jnp.float32)),
      mesh=plsc.VectorSubcoreMesh(core_axis_name="c", subcore_axis_name="s"),
      compiler_params=pltpu.CompilerParams(needs_layout_passes=False),
      scratch_types=[
          pltpu.VMEM((_CHW, _CPN * L), jnp.int32),
          pltpu.VMEM((_CHW, 2 * _CPN), jnp.int32),
          pltpu.VMEM((_CHW, _CPN), jnp.int32),
          pltpu.VMEM((2, _CPN * L, D), jnp.float32),
          pltpu.VMEM((20 * D,), jnp.float32),
          pltpu.VMEM((2, _CPN, D), jnp.float32),
          pltpu.VMEM((2, _CPN, D), jnp.float32),
          pltpu.SemaphoreType.DMA,
          pltpu.SemaphoreType.DMA,
          pltpu.SemaphoreType.DMA,
          pltpu.SemaphoreType.DMA,
      ],
  )
  return f(x3, nt3, perm3, st_table, nt_table)


_ECH = 125
_ECHN = 80
_ZROWS = 200
_NZCH = N // _ZROWS


def _edge_body(src_hbm, dst_hbm, remb_hbm, agg_out,
               sidx_all, didx_all, rows2, agg_sh, semg0, semg1):
  cid = lax.axis_index("c")
  sid = lax.axis_index("s")
  wid = sid * NC + cid
  my_z = (_NZCH - 1 - sid) // NS + 1
  semg = (semg0, semg1)

  zeros16 = jnp.zeros((16,), jnp.float32)

  def zb(r, carry):
    for k in range(D // 16):
      rows2[0, r, pl.ds(k * 16, 16)] = zeros16
    return carry

  lax.fori_loop(0, 128, zb, 0)

  my_zc = (78 - sid) // NS + 1

  def zcopy(i, carry):
    off = pl.multiple_of(jnp.minimum((sid + i * NS) * 128, N - 128), 8)
    pltpu.sync_copy(rows2.at[0], agg_sh.at[pl.ds(off, 128)])
    return carry

  lax.fori_loop(0, my_zc, zcopy, 0)
  plsc.subcore_barrier()

  def start(i, s):
    pltpu.async_copy(remb_hbm.at[sidx_all.at[i]],
                     rows2.at[s, pl.ds(0, _ECH)], semg[s])

  def wait(i, s):
    pltpu.make_async_copy(remb_hbm.at[sidx_all.at[i]],
                          rows2.at[s, pl.ds(0, _ECH)], semg[s]).wait()

  for half in range(2):
    pltpu.sync_copy(src_hbm.at[wid, half], sidx_all)
    pltpu.sync_copy(dst_hbm.at[wid, half], didx_all)
    start(0, 0)

    def eb(j, carry):
      i0 = 2 * j
      i1 = 2 * j + 1
      start(i1, 1)
      wait(i0, 0)
      pltpu.sync_copy(rows2.at[0, pl.ds(0, _ECH)],
                      agg_sh.at[didx_all.at[i0]], add=True)

      @pl.when(j < _ECHN // 4 - 1)
      def _():
        start(i0 + 2, 0)

      wait(i1, 1)
      pltpu.sync_copy(rows2.at[1, pl.ds(0, _ECH)],
                      agg_sh.at[didx_all.at[i1]], add=True)
      return carry

    lax.fori_loop(0, _ECHN // 4, eb, 0)
  plsc.subcore_barrier()

  def wcopy(i, carry):
    off = pl.multiple_of((sid + i * NS) * _ZROWS, 8)
    pltpu.sync_copy(agg_sh.at[pl.ds(off, _ZROWS)],
                    agg_out.at[pl.ds(cid * N + off, _ZROWS)])
    return carry

  lax.fori_loop(0, my_z, wcopy, 0)


def _sc_edges(src3, dst3, relu_emb):
  f = pl.kernel(
      _edge_body,
      out_type=jax.ShapeDtypeStruct((NC * N, D), jnp.float32),
      mesh=plsc.VectorSubcoreMesh(core_axis_name="c", subcore_axis_name="s"),
      compiler_params=pltpu.CompilerParams(needs_layout_passes=False),
      scratch_types=[
          pltpu.VMEM((_ECHN // 2, _ECH), jnp.int32),
          pltpu.VMEM((_ECHN // 2, _ECH), jnp.int32),
          pltpu.VMEM((2, 128, D), jnp.float32),
          pltpu.VMEM_SHARED((N, D), jnp.float32),
          pltpu.SemaphoreType.DMA,
          pltpu.SemaphoreType.DMA,
      ],
  )
  return f(src3, dst3, relu_emb)


def _tc_body(emb_ref, agg_ref, w1_ref, b1_ref, g_ref, be_ref, w2_ref, b2_ref,
             wc_ref, w3_ref, b3_ref, out_ref):
  h = emb_ref[0:N, :] + agg_ref[0:N, :] + agg_ref[N:2 * N, :]
  h1 = jnp.dot(h, w1_ref[...], preferred_element_type=jnp.float32) + b1_ref[...]
  mu = jnp.sum(h1, axis=0, keepdims=True) * (1.0 / N)
  dlt = h1 - mu
  var = jnp.sum(dlt * dlt, axis=0, keepdims=True) * (1.0 / N)
  h1n = dlt / jnp.sqrt(var + BN_EPS) * g_ref[...] + be_ref[...]
  h1n = jnp.maximum(h1n, 0.0)
  xh = jnp.dot(h1n, w2_ref[...], preferred_element_type=jnp.float32) + b2_ref[...]

  wc = wc_ref[...]
  s_flat = jnp.dot(xh, wc, preferred_element_type=jnp.float32) \
      / jnp.sqrt(jnp.sum(wc * wc))

  s2 = jnp.concatenate([s_flat[n * B:(n + 1) * B, :] for n in range(NPG)],
                       axis=1)
  lane = lax.broadcasted_iota(jnp.int32, (B, NPG), 1)
  rank = jnp.zeros((B, NPG), jnp.int32)
  for m in range(NPG):
    cm = s2[:, m:m + 1]
    rank = rank + jnp.where(cm > s2, 1, 0) \
                + jnp.where((cm == s2) & (lane > m), 1, 0)
  sel = rank < K
  t2 = jnp.tanh(s2)

  neg = jnp.float32(-3.0e38)
  gmax = jnp.full((B, H), neg, jnp.float32)
  gsum = jnp.zeros((B, H), jnp.float32)
  for n in range(NPG):
    xn = xh[n * B:(n + 1) * B, :]
    xpn = xn * t2[:, n:n + 1]
    mn = sel[:, n:n + 1]
    gmax = jnp.maximum(gmax, jnp.where(mn, xpn, neg))
    gsum = gsum + jnp.where(mn, xpn, 0.0)
  stmt = jnp.concatenate([gmax, gsum * (1.0 / K)], axis=1)
  out_ref[...] = jnp.dot(stmt, w3_ref[...],
                         preferred_element_type=jnp.float32) + b3_ref[...]


def _tc_dense(emb, aggp, W1, b1, gamma, beta, W2, b2, pool_w, W3, b3):
  return pl.pallas_call(
      _tc_body,
      out_shape=jax.ShapeDtypeStruct((B, H), jnp.float32),
  )(emb, aggp, W1, b1.reshape(1, -1), gamma.reshape(1, -1),
    beta.reshape(1, -1), W2, b2.reshape(1, -1), pool_w.reshape(D, 1),
    W3, b3.reshape(1, -1))


def kernel(x, node_type, edge_index, batch, st_table, nt_table, W1, b1,
           gamma, beta, W2, b2, pool_w, W3, b3):
  del batch
  x_flat = x.astype(jnp.int32).reshape(N * L)
  pad_ids = (jnp.arange(_NPAD * L - N * L, dtype=jnp.int32) * 997) % 99991
  x3 = jnp.concatenate([x_flat, pad_ids]).reshape(NW * _CHW, _CPN * L)
  nt3 = jnp.concatenate(
      [node_type.astype(jnp.int32), jnp.zeros((_NPAD - N,), jnp.int32)]
  ).reshape(NW * _CHW, _CPN)
  nt3 = jnp.pad(nt3, ((0, 0), (0, _CPN)))
  v = jnp.arange(_NPAD, dtype=jnp.int32)
  perm = jnp.where(v < N, (v % NPG) * B + v // NPG, v)
  perm3 = perm.reshape(NW * _CHW, _CPN)
  src = edge_index[0].astype(jnp.int32)
  dst = edge_index[1].astype(jnp.int32)
  src3 = ((src % NPG) * B + src // NPG).reshape(NW, 2, _ECHN // 2, _ECH)
  dst3 = ((dst % NPG) * B + dst // NPG).reshape(NW, 2, _ECHN // 2, _ECH)

  emb, relu_emb = _sc_embed(x3, nt3, perm3, st_table, nt_table.reshape(-1))
  aggp = _sc_edges(src3, dst3, relu_emb)
  return _tc_dense(emb, aggp, W1, b1, gamma, beta, W2, b2, pool_w, W3, b3)
